# baseline (speedup 1.0000x reference)
import jax
import jax.numpy as jnp
from jax import lax
from jax.experimental import pallas as pl
from jax.experimental.pallas import tpu as pltpu
from jax.experimental.pallas import tpu_sc as plsc

BATCH = 4096
SEQ = 20
DIM = 64
NUM_CORES = 2
NUM_SUBCORES = 16
NW = NUM_CORES * NUM_SUBCORES
B_PER_W = BATCH // NW
CHUNK_B = 16
N_CHUNKS = B_PER_W // CHUNK_B
ROWS_PER_CHUNK = CHUNK_B * SEQ
IDS_PER_W = B_PER_W * SEQ
GATHER_SPLITS = ((0, 128), (128, 128), (256, 64))


def _body(ids0_hbm, ids1_hbm, len0_hbm, len1_hbm, t0_hbm, t1_hbm, out_hbm,
          idx0_v, idx1_v, a0_v, b0_v, a1_v, b1_v, len0_v, len1_v,
          gsem0, gsem1, wsem0, wsem1):
  cid = lax.axis_index("c")
  sid = lax.axis_index("s")
  wid = sid * NUM_CORES + cid
  b0w = wid * B_PER_W

  pltpu.sync_copy(len0_hbm.at[pl.ds(b0w, B_PER_W)], len0_v)
  pltpu.sync_copy(len1_hbm.at[pl.ds(b0w, B_PER_W)], len1_v)
  pltpu.sync_copy(ids0_hbm.at[pl.ds(wid * IDS_PER_W, IDS_PER_W)], idx0_v)
  pltpu.sync_copy(ids1_hbm.at[pl.ds(wid * IDS_PER_W, IDS_PER_W)], idx1_v)

  zero = jnp.zeros((16,), jnp.float32)
  bufs = ((a0_v, b0_v), (a1_v, b1_v))
  gsems = (gsem0, gsem1)
  wsems = (wsem0, wsem1)

  def fire_gathers(c, p):
    a_v, b_v = bufs[p]
    base = c * ROWS_PER_CHUNK
    for off, n in GATHER_SPLITS:
      sl = pl.ds(off, n)
      pltpu.async_copy(t0_hbm.at[idx0_v.at[pl.ds(base + off, n)]],
                       a_v.at[sl], gsems[p])
      pltpu.async_copy(t1_hbm.at[idx1_v.at[pl.ds(base + off, n)]],
                       b_v.at[sl], gsems[p])

  def drain_gathers(p):
    a_v, b_v = bufs[p]
    pltpu.make_async_copy(t0_hbm.at[pl.ds(0, ROWS_PER_CHUNK)], a_v,
                          gsems[p]).wait()
    pltpu.make_async_copy(t1_hbm.at[pl.ds(0, ROWS_PER_CHUNK)], b_v,
                          gsems[p]).wait()

  def drain_writes(p):
    a_v, b_v = bufs[p]
    pltpu.make_async_copy(out_hbm.at[pl.ds(0, ROWS_PER_CHUNK)], a_v,
                          wsems[p]).wait()
    pltpu.make_async_copy(out_hbm.at[pl.ds(0, ROWS_PER_CHUNK)], b_v,
                          wsems[p]).wait()

  def zero_tail(ref, base_row, start):
    def zrow(s, _):
      r = base_row + s
      ref[r, pl.ds(0, 16)] = zero
      ref[r, pl.ds(16, 16)] = zero
      ref[r, pl.ds(32, 16)] = zero
      ref[r, pl.ds(48, 16)] = zero
      return 0
    lax.fori_loop(start, SEQ, zrow, 0)

  def process(c, p):
    a_v, b_v = bufs[p]
    lv0 = len0_v[pl.ds(c * CHUNK_B, CHUNK_B)]
    lv1 = len1_v[pl.ds(c * CHUNK_B, CHUNK_B)]
    for lane in range(CHUNK_B):
      zero_tail(a_v, lane * SEQ, lv0[lane])
      zero_tail(b_v, lane * SEQ, lv1[lane])
    for bi in range(CHUNK_B):
      gb = b0w + c * CHUNK_B + bi
      src = pl.ds(bi * SEQ, SEQ)
      pltpu.async_copy(a_v.at[src], out_hbm.at[pl.ds(gb * 2 * SEQ, SEQ)],
                       wsems[p])
      pltpu.async_copy(b_v.at[src], out_hbm.at[pl.ds(gb * 2 * SEQ + SEQ, SEQ)],
                       wsems[p])

  fire_gathers(0, 0)
  fire_gathers(1, 1)
  for c in range(N_CHUNKS):
    p = c % 2
    if c >= 2:
      drain_writes(p)
      fire_gathers(c, p)
    drain_gathers(p)
    process(c, p)
  drain_writes(0)
  drain_writes(1)


@jax.jit
def _run(ids_f0, ids_f1, lengths_f0, lengths_f1, table_f0, table_f1):
  mesh = plsc.VectorSubcoreMesh(core_axis_name="c", subcore_axis_name="s")
  ids0 = ids_f0.reshape(BATCH * SEQ)
  ids1 = ids_f1.reshape(BATCH * SEQ)
  out = pl.kernel(
      _body,
      out_type=jax.ShapeDtypeStruct((BATCH * 2 * SEQ, DIM), jnp.float32),
      mesh=mesh,
      compiler_params=pltpu.CompilerParams(use_tc_tiling_on_sc=False),
      scratch_types=[
          pltpu.VMEM((IDS_PER_W,), jnp.int32),
          pltpu.VMEM((IDS_PER_W,), jnp.int32),
          pltpu.VMEM((ROWS_PER_CHUNK, DIM), jnp.float32),
          pltpu.VMEM((ROWS_PER_CHUNK, DIM), jnp.float32),
          pltpu.VMEM((ROWS_PER_CHUNK, DIM), jnp.float32),
          pltpu.VMEM((ROWS_PER_CHUNK, DIM), jnp.float32),
          pltpu.VMEM((B_PER_W,), jnp.int32),
          pltpu.VMEM((B_PER_W,), jnp.int32),
          pltpu.SemaphoreType.DMA,
          pltpu.SemaphoreType.DMA,
          pltpu.SemaphoreType.DMA,
          pltpu.SemaphoreType.DMA,
      ],
  )(ids0, ids1, lengths_f0, lengths_f1, table_f0, table_f1)
  return out.reshape(BATCH, 2 * SEQ * DIM)


def kernel(ids_f0, ids_f1, lengths_f0, lengths_f1, table_f0, table_f1):
  return _run(ids_f0, ids_f1, lengths_f0, lengths_f1, table_f0, table_f1)
